# baseline (device time: 43405 ns/iter reference)
import jax
import jax.numpy as jnp
from jax import lax
from jax.experimental import pallas as pl
from jax.experimental.pallas import tpu as pltpu

N_DEV = 8


def kernel(Q, K, V):
    b, sq, h, d = Q.shape
    skv = K.shape[1]
    scale = d ** -0.5

    def body(q_ref, k_ref, v_ref, out_ref, mine_ref, comm_ref, send_sems, recv_sems):
        my = lax.axis_index("i")

        q = q_ref[:].astype(jnp.float32)
        k = k_ref[:].astype(jnp.float32)
        v = v_ref[:].astype(jnp.float32)
        s = jnp.sum(k * q, axis=-1) * scale
        p = jnp.exp(s)
        den = jnp.sum(p, axis=1)
        num = jnp.sum(p[:, :, :, None] * v, axis=1)

        mine_ref[0] = num
        mine_ref[1] = jnp.broadcast_to(den[:, :, None], (b, h, d))

        rdmas = []
        for off in range(1, N_DEV):
            tgt = (my + off) % N_DEV
            rdma = pltpu.make_async_remote_copy(
                src_ref=mine_ref,
                dst_ref=comm_ref.at[off - 1],
                send_sem=send_sems.at[off - 1],
                recv_sem=recv_sems.at[off - 1],
                device_id=(tgt,),
                device_id_type=pl.DeviceIdType.MESH,
            )
            rdma.start()
            rdmas.append(rdma)

        for rdma in rdmas:
            rdma.wait_recv()

        acc = mine_ref[:]
        for slot in range(N_DEV - 1):
            acc = acc + comm_ref[slot]
        o = acc[0] / acc[1]
        out_ref[:] = o.reshape(b, sq, h, d)

        for rdma in rdmas:
            rdma.wait_send()

    return pl.pallas_call(
        body,
        out_shape=jax.ShapeDtypeStruct((b, sq, h, d), jnp.float32),
        in_specs=[
            pl.BlockSpec(memory_space=pltpu.VMEM),
            pl.BlockSpec(memory_space=pltpu.VMEM),
            pl.BlockSpec(memory_space=pltpu.VMEM),
        ],
        out_specs=pl.BlockSpec(memory_space=pltpu.VMEM),
        scratch_shapes=[
            pltpu.VMEM((2, b, h, d), jnp.float32),
            pltpu.VMEM((N_DEV - 1, 2, b, h, d), jnp.float32),
            pltpu.SemaphoreType.DMA((N_DEV - 1,)),
            pltpu.SemaphoreType.DMA((N_DEV - 1,)),
        ],
    )(Q, K, V)
